# baseline (speedup 1.0000x reference)
import functools

import jax
import jax.numpy as jnp
from jax import lax
from jax.experimental import pallas as pl
from jax.experimental.pallas import tpu as pltpu
from jax.experimental.pallas import tpu_sc as plsc

MAX_STEPS = 8
STOP_RELATION = -1
DIRECTION_FORWARD = 0

_LANES = 16


@functools.lru_cache(maxsize=None)
def _make_sc_segmin(B, per_n, sentinel):
    mesh = plsc.VectorSubcoreMesh(core_axis_name="c", subcore_axis_name="s")

    @functools.partial(
        pl.kernel,
        mesh=mesh,
        compiler_params=pltpu.CompilerParams(needs_layout_passes=False),
        out_type=(
            jax.ShapeDtypeStruct((B, _LANES), jnp.int32),
            jax.ShapeDtypeStruct((B, _LANES), jnp.int32),
            jax.ShapeDtypeStruct((B, _LANES), jnp.int32),
            jax.ShapeDtypeStruct((B, _LANES), jnp.int32),
        ),
        scratch_types=[
            pltpu.VMEM((per_n,), jnp.int32),
            pltpu.VMEM((per_n,), jnp.int32),
            pltpu.VMEM((_LANES,), jnp.int32),
            pltpu.VMEM((_LANES,), jnp.int32),
        ],
    )
    def sc_segmin(start_hbm, answer_hbm, extra_hbm,
                  minl_hbm, hits_hbm, ans_hbm, done_hbm,
                  s_v, a_v, stage_v, extra_v):
        c = lax.axis_index("c")
        s = lax.axis_index("s")

        @pl.when(c == 0)
        def _scan():
            base = s * per_n
            pltpu.sync_copy(start_hbm.at[pl.ds(base, per_n)], s_v)
            pltpu.sync_copy(answer_hbm.at[pl.ds(base, per_n)], a_v)
            pltpu.sync_copy(extra_hbm, extra_v)

            def body(i, acc):
                off = i * _LANES
                vs = s_v[pl.ds(off, _LANES)]
                va = a_v[pl.ds(off, _LANES)]
                idx = lax.iota(jnp.int32, _LANES) + off
                hit = (vs > 0) & (va > 0)
                return jnp.minimum(acc, jnp.where(hit, idx, sentinel))

            acc = lax.fori_loop(
                0, per_n // _LANES, body,
                jnp.full((_LANES,), sentinel, jnp.int32), unroll=8)

            dnums = lax.GatherDimensionNumbers(
                offset_dims=(), collapsed_slice_dims=(0,),
                start_index_map=(0,))
            for off in (8, 4, 2, 1):
                perm = (lax.iota(jnp.int32, _LANES) + off) & (_LANES - 1)
                rot = lax.gather(
                    acc, perm[:, None], dimension_numbers=dnums,
                    slice_sizes=(1,),
                    mode=lax.GatherScatterMode.PROMISE_IN_BOUNDS)
                acc = jnp.minimum(acc, rot)

            hit_mask = acc != sentinel
            hits_i = hit_mask.astype(jnp.int32)
            ansh = jnp.where(hit_mask, acc, -1)
            ev = extra_v[...]
            perm_s = jnp.broadcast_to(s, (_LANES,)).astype(jnp.int32)
            extra_b = lax.gather(
                ev, perm_s[:, None], dimension_numbers=dnums,
                slice_sizes=(1,),
                mode=lax.GatherScatterMode.PROMISE_IN_BOUNDS)
            done_i = jnp.maximum(hits_i, extra_b)

            stage_v[...] = acc
            pltpu.sync_copy(stage_v, minl_hbm.at[s])
            stage_v[...] = hits_i
            pltpu.sync_copy(stage_v, hits_hbm.at[s])
            stage_v[...] = ansh
            pltpu.sync_copy(stage_v, ans_hbm.at[s])
            stage_v[...] = done_i
            pltpu.sync_copy(stage_v, done_hbm.at[s])

    return sc_segmin


@functools.lru_cache(maxsize=None)
def _make_tc_assemble(N, D, E, B, rows_per_blk):
    grid = (N // rows_per_blk,)

    def body(nt_in, nis_in, q_in,
             nt_out, act_out, vis_out, uem_out, q_out,
             actions_out, dirs_out, sc_out):
        i = pl.program_id(0)
        nt_out[...] = nt_in[...]

        @pl.when(i == 0)
        def _():
            act_out[...] = nis_in[...]
            vis_out[...] = nis_in[...]
            uem_out[...] = jnp.zeros((E,), dtype=jnp.bool_)
            q_out[...] = q_in[...]
            actions_out[...] = jnp.full(
                (B, MAX_STEPS + 1), STOP_RELATION, dtype=jnp.int32)
            dirs_out[...] = jnp.full(
                (B, MAX_STEPS + 1), DIRECTION_FORWARD, dtype=jnp.int32)
            sc_out[...] = jnp.zeros((B,), dtype=jnp.int32)

    blk = pl.BlockSpec((rows_per_blk, D), lambda i: (i, 0))
    whole1d_n = pl.BlockSpec((N,), lambda i: (0,))
    whole1d_e = pl.BlockSpec((E,), lambda i: (0,))
    whole1d_b = pl.BlockSpec((B,), lambda i: (0,))
    whole2d_q = pl.BlockSpec((B, D), lambda i: (0, 0))
    whole2d_a = pl.BlockSpec((B, MAX_STEPS + 1), lambda i: (0, 0))

    return pl.pallas_call(
        body,
        grid=grid,
        in_specs=[blk, whole1d_n, whole2d_q],
        out_specs=[blk, whole1d_n, whole1d_n, whole1d_e, whole2d_q,
                   whole2d_a, whole2d_a, whole1d_b],
        out_shape=[
            jax.ShapeDtypeStruct((N, D), jnp.float32),
            jax.ShapeDtypeStruct((N,), jnp.bool_),
            jax.ShapeDtypeStruct((N,), jnp.bool_),
            jax.ShapeDtypeStruct((E,), jnp.bool_),
            jax.ShapeDtypeStruct((B, D), jnp.float32),
            jax.ShapeDtypeStruct((B, MAX_STEPS + 1), jnp.int32),
            jax.ShapeDtypeStruct((B, MAX_STEPS + 1), jnp.int32),
            jax.ShapeDtypeStruct((B,), jnp.int32),
        ],
        compiler_params=pltpu.CompilerParams(
            dimension_semantics=("arbitrary",)),
    )


def kernel(edge_index, edge_batch, edge_relations, question_tokens, node_tokens,
           node_ptr, edge_ptr, start_node_locals, start_ptr,
           answer_node_locals, answer_ptr, dummy_mask,
           node_batch, node_in_degree, node_is_start, node_is_answer):
    B = int(node_ptr.shape[0]) - 1
    N = int(node_is_start.shape[0])
    E = int(edge_index.shape[1])
    per_n = N // B
    sentinel = N + 1

    question_tokens = question_tokens.astype(jnp.float32)
    node_tokens = node_tokens.astype(jnp.float32)

    start_i = node_is_start.astype(jnp.int32)
    answer_i = node_is_answer.astype(jnp.int32)
    start_counts = start_ptr[1:] - start_ptr[:-1]
    extra_i = ((start_counts == 0) | dummy_mask).astype(jnp.int32)

    sc_segmin = _make_sc_segmin(B, per_n, sentinel)
    _minl16, hits16, ans16, done16 = sc_segmin(start_i, answer_i, extra_i)

    answer_hits = hits16[:, 0].astype(bool)
    done = done16[:, 0].astype(bool)
    answer_node_hit = ans16[:, 0]
    start_node_hit = answer_node_hit

    D = int(node_tokens.shape[1])
    tc_assemble = _make_tc_assemble(N, D, E, B, 2048)
    (node_tokens, active_nodes, visited_nodes, used_edge_mask,
     question_tokens, actions, directions, step_counts) = tc_assemble(
        node_tokens, node_is_start, question_tokens)

    return (active_nodes, visited_nodes, used_edge_mask, actions, directions,
            done, step_counts, answer_hits, answer_node_hit, start_node_hit,
            node_tokens, question_tokens)
